# Initial kernel scaffold; baseline (speedup 1.0000x reference)
import jax
import jax.numpy as jnp
from jax.experimental import pallas as pl


def kernel(edges, node_emb, rel_emb, W1, b1, W2, b2, Ws1, bs1, Ws2, bs2, g1, be1, g2, be2):
    raise NotImplementedError("write your pallas kernel here")


# speedup vs baseline: 4.7902x; 4.7902x over previous
import jax
import jax.numpy as jnp
from jax import lax
from jax.experimental import pallas as pl
from jax.experimental.pallas import tpu as pltpu
from jax.experimental.pallas import tpu_sc as plsc

N = 10000
D = 128
E = 320000
NC = 2
NS = 16
NW = NC * NS
NP = 10240
EPW = E // NW
C = 80
NCHUNK = EPW // C
CW = 128
SB = 64
RPT = NP // NS


def _sc_agg_body(x_hbm, src_hbm, dst_hbm, zrow_hbm, out0, out1,
                 sidx, didx, rows, stage, agg_sh, gsem):
    cid = lax.axis_index("c")
    sid = lax.axis_index("s")
    wid = sid * NC + cid
    e_base = wid * EPW

    row0 = sid * RPT
    pltpu.sync_copy(zrow_hbm, stage)
    for part in range(RPT // SB):
        pltpu.sync_copy(stage, agg_sh.at[pl.ds(row0 + part * SB, SB)])
    plsc.subcore_barrier()

    @pl.loop(0, NCHUNK)
    def _(j):
        base = e_base + j * C
        pltpu.sync_copy(src_hbm.at[pl.ds(base, C)], sidx)
        pltpu.sync_copy(dst_hbm.at[pl.ds(base, C)], didx)
        pltpu.async_copy(x_hbm.at[sidx], rows, gsem).wait()
        pltpu.sync_copy(rows, agg_sh.at[didx], add=True)

    plsc.subcore_barrier()

    @pl.when(cid == 0)
    def _():
        for part in range(RPT // SB):
            r = row0 + part * SB
            pltpu.sync_copy(agg_sh.at[pl.ds(r, SB)], stage)
            pltpu.sync_copy(stage, out0.at[pl.ds(r, SB)])

    @pl.when(cid == 1)
    def _():
        for part in range(RPT // SB):
            r = row0 + part * SB
            pltpu.sync_copy(agg_sh.at[pl.ds(r, SB)], stage)
            pltpu.sync_copy(stage, out1.at[pl.ds(r, SB)])


def _make_sc_agg():
    mesh = plsc.VectorSubcoreMesh(core_axis_name="c", subcore_axis_name="s")
    return pl.kernel(
        _sc_agg_body,
        out_type=(jax.ShapeDtypeStruct((NP, D), jnp.float32),
                  jax.ShapeDtypeStruct((NP, D), jnp.float32)),
        mesh=mesh,
        scratch_types=(
            pltpu.VMEM((C,), jnp.int32),
            pltpu.VMEM((C,), jnp.int32),
            pltpu.VMEM((C, D), jnp.float32),
            pltpu.VMEM((SB, D), jnp.float32),
            pltpu.VMEM_SHARED((NP, D), jnp.float32),
            pltpu.SemaphoreType.DMA,
        ),
    )


def _sc_cnt_body(dst_hbm, zcnt_hbm, ones_hbm, cnt0, cnt1,
                 didx, ones_v, cstage, cnt_sh):
    cid = lax.axis_index("c")
    sid = lax.axis_index("s")
    wid = sid * NC + cid
    e_base = wid * EPW

    row0 = sid * RPT
    pltpu.sync_copy(zcnt_hbm, cstage)
    pltpu.sync_copy(ones_hbm, ones_v)
    for part in range(RPT // SB):
        pltpu.sync_copy(cstage, cnt_sh.at[pl.ds(row0 + part * SB, SB)])
    plsc.subcore_barrier()

    @pl.loop(0, NCHUNK)
    def _(j):
        base = e_base + j * C
        pltpu.sync_copy(dst_hbm.at[pl.ds(base, C)], didx)
        pltpu.sync_copy(ones_v, cnt_sh.at[didx], add=True)

    plsc.subcore_barrier()

    @pl.when(cid == 0)
    def _():
        for part in range(RPT // SB):
            r = row0 + part * SB
            pltpu.sync_copy(cnt_sh.at[pl.ds(r, SB)], cstage)
            pltpu.sync_copy(cstage, cnt0.at[pl.ds(r, SB)])

    @pl.when(cid == 1)
    def _():
        for part in range(RPT // SB):
            r = row0 + part * SB
            pltpu.sync_copy(cnt_sh.at[pl.ds(r, SB)], cstage)
            pltpu.sync_copy(cstage, cnt1.at[pl.ds(r, SB)])


def _make_sc_cnt():
    mesh = plsc.VectorSubcoreMesh(core_axis_name="c", subcore_axis_name="s")
    return pl.kernel(
        _sc_cnt_body,
        out_type=(jax.ShapeDtypeStruct((NP, CW), jnp.float32),
                  jax.ShapeDtypeStruct((NP, CW), jnp.float32)),
        mesh=mesh,
        scratch_types=(
            pltpu.VMEM((C,), jnp.int32),
            pltpu.VMEM((C, CW), jnp.float32),
            pltpu.VMEM((SB, CW), jnp.float32),
            pltpu.VMEM_SHARED((NP, CW), jnp.float32),
        ),
    )


def _dense_block(p0, p1, c0, c1, x, W, b, Ws, bs, g, be, o_ref):
    agg = p0[...] + p1[...]
    cnt = c0[:, 0:1] + c1[:, 0:1]
    mean = agg / jnp.maximum(cnt, 1.0)
    dn = (((1,), (1,)), ((), ()))
    h = lax.dot_general(mean, W[...], dn, preferred_element_type=jnp.float32)
    h = h + b[...] * jnp.where(cnt > 0.0, 1.0, 0.0)
    h = jnp.maximum(h, 0.0)
    o = h + lax.dot_general(x[...], Ws[...], dn,
                            preferred_element_type=jnp.float32) + bs[...]
    m = jnp.mean(o, axis=-1, keepdims=True)
    v = jnp.mean((o - m) * (o - m), axis=-1, keepdims=True)
    o_ref[...] = (o - m) * lax.rsqrt(v + 1e-5) * g[...] + be[...]


BR = 1024


def _dense(p0, p1, c0, c1, x, W, b, Ws, bs, g, be):
    row_spec = pl.BlockSpec((BR, D), lambda i: (i, 0))
    cnt_spec = pl.BlockSpec((BR, CW), lambda i: (i, 0))
    mat_spec = pl.BlockSpec((D, D), lambda i: (0, 0))
    vec_spec = pl.BlockSpec((1, D), lambda i: (0, 0))
    return pl.pallas_call(
        _dense_block,
        grid=(NP // BR,),
        in_specs=[row_spec, row_spec, cnt_spec, cnt_spec, row_spec,
                  mat_spec, vec_spec, mat_spec, vec_spec, vec_spec, vec_spec],
        out_specs=row_spec,
        out_shape=jax.ShapeDtypeStruct((NP, D), jnp.float32),
    )(p0, p1, c0, c1, x, W, b.reshape(1, D), Ws, bs.reshape(1, D),
      g.reshape(1, D), be.reshape(1, D))


def kernel(edges, node_emb, rel_emb, W1, b1, W2, b2, Ws1, bs1, Ws2, bs2,
           g1, be1, g2, be2):
    src = edges[:, 0]
    dst = edges[:, 2]
    x = jnp.pad(node_emb, ((0, NP - N), (0, 0)))
    zrow = jnp.zeros((SB, D), jnp.float32)
    zcnt = jnp.zeros((SB, CW), jnp.float32)
    ones = jnp.ones((C, CW), jnp.float32)

    sc_agg = _make_sc_agg()
    c0, c1 = _make_sc_cnt()(dst, zcnt, ones)
    a0, a1 = sc_agg(x, src, dst, zrow)
    x1 = _dense(a0, a1, c0, c1, x, W1, b1, Ws1, bs1, g1, be1)
    b0, b1_ = sc_agg(x1, src, dst, zrow)
    x2 = _dense(b0, b1_, c0, c1, x1, W2, b2, Ws2, bs2, g2, be2)
    return x2[:N]


# speedup vs baseline: 6.8618x; 1.4325x over previous
import jax
import jax.numpy as jnp
from jax import lax
from jax.experimental import pallas as pl
from jax.experimental.pallas import tpu as pltpu
from jax.experimental.pallas import tpu_sc as plsc

N = 10000
D = 128
E = 320000
NC = 2
NS = 16
NW = NC * NS
NP = 10240
EPW = E // NW
C = 80
NCHUNK = EPW // C
CW = 128
SB = 32
RPT = NP // NS


def _sc_agg_body(x_hbm, src_hbm, dst_hbm, zrow_hbm, out0, out1,
                 sidx, didx, rows, stage, agg_sh, gsem):
    cid = lax.axis_index("c")
    sid = lax.axis_index("s")
    wid = sid * NC + cid

    row0 = sid * RPT
    isem = pltpu.async_copy(src_hbm.at[wid], sidx, gsem)
    pltpu.sync_copy(dst_hbm.at[wid], didx)
    isem.wait()
    pltpu.sync_copy(zrow_hbm, stage)
    for part in range(RPT // SB):
        pltpu.sync_copy(stage, agg_sh.at[pl.ds(row0 + part * SB, SB)])
    plsc.subcore_barrier()

    @pl.loop(0, NCHUNK)
    def _(j):
        pltpu.async_copy(x_hbm.at[sidx.at[j]], rows, gsem).wait()
        pltpu.sync_copy(rows, agg_sh.at[didx.at[j]], add=True)

    plsc.subcore_barrier()

    @pl.when(cid == 0)
    def _():
        for part in range(RPT // SB):
            r = row0 + part * SB
            pltpu.sync_copy(agg_sh.at[pl.ds(r, SB)], stage)
            pltpu.sync_copy(stage, out0.at[pl.ds(r, SB)])

    @pl.when(cid == 1)
    def _():
        for part in range(RPT // SB):
            r = row0 + part * SB
            pltpu.sync_copy(agg_sh.at[pl.ds(r, SB)], stage)
            pltpu.sync_copy(stage, out1.at[pl.ds(r, SB)])


def _make_sc_agg():
    mesh = plsc.VectorSubcoreMesh(core_axis_name="c", subcore_axis_name="s")
    return pl.kernel(
        _sc_agg_body,
        out_type=(jax.ShapeDtypeStruct((NP, D), jnp.float32),
                  jax.ShapeDtypeStruct((NP, D), jnp.float32)),
        mesh=mesh,
        scratch_types=(
            pltpu.VMEM((NCHUNK, C), jnp.int32),
            pltpu.VMEM((NCHUNK, C), jnp.int32),
            pltpu.VMEM((C, D), jnp.float32),
            pltpu.VMEM((SB, D), jnp.float32),
            pltpu.VMEM_SHARED((NP, D), jnp.float32),
            pltpu.SemaphoreType.DMA,
        ),
    )


def _sc_cnt_body(dst_hbm, zcnt_hbm, ones_hbm, cnt0, cnt1,
                 didx, ones_v, cstage, cnt_sh):
    cid = lax.axis_index("c")
    sid = lax.axis_index("s")
    wid = sid * NC + cid

    row0 = sid * RPT
    pltpu.sync_copy(dst_hbm.at[wid], didx)
    pltpu.sync_copy(zcnt_hbm, cstage)
    pltpu.sync_copy(ones_hbm, ones_v)
    for part in range(RPT // SB):
        pltpu.sync_copy(cstage, cnt_sh.at[pl.ds(row0 + part * SB, SB)])
    plsc.subcore_barrier()

    @pl.loop(0, NCHUNK)
    def _(j):
        pltpu.sync_copy(ones_v, cnt_sh.at[didx.at[j]], add=True)

    plsc.subcore_barrier()

    @pl.when(cid == 0)
    def _():
        for part in range(RPT // SB):
            r = row0 + part * SB
            pltpu.sync_copy(cnt_sh.at[pl.ds(r, SB)], cstage)
            pltpu.sync_copy(cstage, cnt0.at[pl.ds(r, SB)])

    @pl.when(cid == 1)
    def _():
        for part in range(RPT // SB):
            r = row0 + part * SB
            pltpu.sync_copy(cnt_sh.at[pl.ds(r, SB)], cstage)
            pltpu.sync_copy(cstage, cnt1.at[pl.ds(r, SB)])


def _make_sc_cnt():
    mesh = plsc.VectorSubcoreMesh(core_axis_name="c", subcore_axis_name="s")
    return pl.kernel(
        _sc_cnt_body,
        out_type=(jax.ShapeDtypeStruct((NP, CW), jnp.float32),
                  jax.ShapeDtypeStruct((NP, CW), jnp.float32)),
        mesh=mesh,
        scratch_types=(
            pltpu.VMEM((NCHUNK, C), jnp.int32),
            pltpu.VMEM((C, CW), jnp.float32),
            pltpu.VMEM((SB, CW), jnp.float32),
            pltpu.VMEM_SHARED((NP, CW), jnp.float32),
        ),
    )


def _dense_block(p0, p1, c0, c1, x, W, b, Ws, bs, g, be, o_ref):
    agg = p0[...] + p1[...]
    cnt = c0[:, 0:1] + c1[:, 0:1]
    mean = agg / jnp.maximum(cnt, 1.0)
    dn = (((1,), (1,)), ((), ()))
    h = lax.dot_general(mean, W[...], dn, preferred_element_type=jnp.float32)
    h = h + b[...] * jnp.where(cnt > 0.0, 1.0, 0.0)
    h = jnp.maximum(h, 0.0)
    o = h + lax.dot_general(x[...], Ws[...], dn,
                            preferred_element_type=jnp.float32) + bs[...]
    m = jnp.mean(o, axis=-1, keepdims=True)
    v = jnp.mean((o - m) * (o - m), axis=-1, keepdims=True)
    o_ref[...] = (o - m) * lax.rsqrt(v + 1e-5) * g[...] + be[...]


BR = 1024


def _dense(p0, p1, c0, c1, x, W, b, Ws, bs, g, be):
    row_spec = pl.BlockSpec((BR, D), lambda i: (i, 0))
    cnt_spec = pl.BlockSpec((BR, CW), lambda i: (i, 0))
    mat_spec = pl.BlockSpec((D, D), lambda i: (0, 0))
    vec_spec = pl.BlockSpec((1, D), lambda i: (0, 0))
    return pl.pallas_call(
        _dense_block,
        grid=(NP // BR,),
        in_specs=[row_spec, row_spec, cnt_spec, cnt_spec, row_spec,
                  mat_spec, vec_spec, mat_spec, vec_spec, vec_spec, vec_spec],
        out_specs=row_spec,
        out_shape=jax.ShapeDtypeStruct((NP, D), jnp.float32),
    )(p0, p1, c0, c1, x, W, b.reshape(1, D), Ws, bs.reshape(1, D),
      g.reshape(1, D), be.reshape(1, D))


def kernel(edges, node_emb, rel_emb, W1, b1, W2, b2, Ws1, bs1, Ws2, bs2,
           g1, be1, g2, be2):
    src = edges[:, 0].reshape(NW, NCHUNK, C)
    dst = edges[:, 2].reshape(NW, NCHUNK, C)
    x = jnp.pad(node_emb, ((0, NP - N), (0, 0)))
    zrow = jnp.zeros((SB, D), jnp.float32)
    zcnt = jnp.zeros((SB, CW), jnp.float32)
    ones = jnp.ones((C, CW), jnp.float32)

    sc_agg = _make_sc_agg()
    c0, c1 = _make_sc_cnt()(dst, zcnt, ones)
    a0, a1 = sc_agg(x, src, dst, zrow)
    x1 = _dense(a0, a1, c0, c1, x, W1, b1, Ws1, bs1, g1, be1)
    b0, b1_ = sc_agg(x1, src, dst, zrow)
    x2 = _dense(b0, b1_, c0, c1, x1, W2, b2, Ws2, bs2, g2, be2)
    return x2[:N]


# speedup vs baseline: 8.4783x; 1.2356x over previous
import jax
import jax.numpy as jnp
from jax import lax
from jax.experimental import pallas as pl
from jax.experimental.pallas import tpu as pltpu
from jax.experimental.pallas import tpu_sc as plsc

N = 10000
D = 128
E = 320000
NC = 2
NS = 16
NW = NC * NS
NP = 10240
EPW = E // NW
C = 80
NCHUNK = EPW // C
CW = 128
SB = 80
RPT = NP // NS


def _sc_agg_body(x_hbm, src_hbm, dst_hbm, zrow_hbm, out0, out1,
                 sidx, didx, rows, agg_sh, semA, semB):
    cid = lax.axis_index("c")
    sid = lax.axis_index("s")
    wid = sid * NC + cid

    row0 = sid * RPT
    isem = pltpu.async_copy(src_hbm.at[wid], sidx, semA)
    pltpu.sync_copy(dst_hbm.at[wid], didx)
    isem.wait()
    pltpu.sync_copy(zrow_hbm, rows.at[0])
    for part in range(RPT // SB):
        pltpu.sync_copy(rows.at[0], agg_sh.at[pl.ds(row0 + part * SB, SB)])
    plsc.subcore_barrier()

    pltpu.async_copy(x_hbm.at[sidx.at[pl.ds(0, C)]], rows.at[0], semA)

    @pl.loop(0, NCHUNK // 2)
    def _(k):
        j0 = 2 * k
        pltpu.make_async_copy(x_hbm.at[sidx.at[pl.ds(j0 * C, C)]],
                              rows.at[0], semA).wait()
        pltpu.async_copy(x_hbm.at[sidx.at[pl.ds((j0 + 1) * C, C)]],
                         rows.at[1], semB)
        pltpu.sync_copy(rows.at[0], agg_sh.at[didx.at[j0]], add=True)
        pltpu.make_async_copy(x_hbm.at[sidx.at[pl.ds((j0 + 1) * C, C)]],
                              rows.at[1], semB).wait()
        pltpu.async_copy(x_hbm.at[sidx.at[pl.ds((j0 + 2) * C, C)]],
                         rows.at[0], semA)
        pltpu.sync_copy(rows.at[1], agg_sh.at[didx.at[j0 + 1]], add=True)

    pltpu.make_async_copy(x_hbm.at[sidx.at[pl.ds((NCHUNK - 1) * C, C)]],
                          rows.at[0], semA).wait()
    pltpu.sync_copy(rows.at[0], agg_sh.at[didx.at[NCHUNK - 1]], add=True)

    plsc.subcore_barrier()

    @pl.when(cid == 0)
    def _():
        for part in range(RPT // SB):
            r = row0 + part * SB
            pltpu.sync_copy(agg_sh.at[pl.ds(r, SB)], rows.at[0])
            pltpu.sync_copy(rows.at[0], out0.at[pl.ds(r, SB)])

    @pl.when(cid == 1)
    def _():
        for part in range(RPT // SB):
            r = row0 + part * SB
            pltpu.sync_copy(agg_sh.at[pl.ds(r, SB)], rows.at[0])
            pltpu.sync_copy(rows.at[0], out1.at[pl.ds(r, SB)])


def _make_sc_agg():
    mesh = plsc.VectorSubcoreMesh(core_axis_name="c", subcore_axis_name="s")
    return pl.kernel(
        _sc_agg_body,
        out_type=(jax.ShapeDtypeStruct((NP, D), jnp.float32),
                  jax.ShapeDtypeStruct((NP, D), jnp.float32)),
        mesh=mesh,
        scratch_types=(
            pltpu.VMEM((EPW,), jnp.int32),
            pltpu.VMEM((NCHUNK, C), jnp.int32),
            pltpu.VMEM((2, C, D), jnp.float32),
            pltpu.VMEM_SHARED((NP, D), jnp.float32),
            pltpu.SemaphoreType.DMA,
            pltpu.SemaphoreType.DMA,
        ),
    )


def _sc_cnt_body(dst_hbm, zcnt_hbm, ones_hbm, cnt0, cnt1,
                 didx, ones_v, cstage, cnt_sh):
    cid = lax.axis_index("c")
    sid = lax.axis_index("s")
    wid = sid * NC + cid

    row0 = sid * RPT
    pltpu.sync_copy(dst_hbm.at[wid], didx)
    pltpu.sync_copy(zcnt_hbm, cstage)
    pltpu.sync_copy(ones_hbm, ones_v)
    for part in range(RPT // SB):
        pltpu.sync_copy(cstage, cnt_sh.at[pl.ds(row0 + part * SB, SB)])
    plsc.subcore_barrier()

    @pl.loop(0, NCHUNK)
    def _(j):
        pltpu.sync_copy(ones_v, cnt_sh.at[didx.at[j]], add=True)

    plsc.subcore_barrier()

    @pl.when(cid == 0)
    def _():
        for part in range(RPT // SB):
            r = row0 + part * SB
            pltpu.sync_copy(cnt_sh.at[pl.ds(r, SB)], cstage)
            pltpu.sync_copy(cstage, cnt0.at[pl.ds(r, SB)])

    @pl.when(cid == 1)
    def _():
        for part in range(RPT // SB):
            r = row0 + part * SB
            pltpu.sync_copy(cnt_sh.at[pl.ds(r, SB)], cstage)
            pltpu.sync_copy(cstage, cnt1.at[pl.ds(r, SB)])


def _make_sc_cnt():
    mesh = plsc.VectorSubcoreMesh(core_axis_name="c", subcore_axis_name="s")
    return pl.kernel(
        _sc_cnt_body,
        out_type=(jax.ShapeDtypeStruct((NP, CW), jnp.float32),
                  jax.ShapeDtypeStruct((NP, CW), jnp.float32)),
        mesh=mesh,
        scratch_types=(
            pltpu.VMEM((NCHUNK, C), jnp.int32),
            pltpu.VMEM((C, CW), jnp.float32),
            pltpu.VMEM((SB, CW), jnp.float32),
            pltpu.VMEM_SHARED((NP, CW), jnp.float32),
        ),
    )


def _dense_block(p0, p1, c0, c1, x, W, b, Ws, bs, g, be, o_ref):
    agg = p0[...] + p1[...]
    cnt = c0[:, 0:1] + c1[:, 0:1]
    mean = agg / jnp.maximum(cnt, 1.0)
    dn = (((1,), (1,)), ((), ()))
    h = lax.dot_general(mean, W[...], dn, preferred_element_type=jnp.float32)
    h = h + b[...] * jnp.where(cnt > 0.0, 1.0, 0.0)
    h = jnp.maximum(h, 0.0)
    o = h + lax.dot_general(x[...], Ws[...], dn,
                            preferred_element_type=jnp.float32) + bs[...]
    m = jnp.mean(o, axis=-1, keepdims=True)
    v = jnp.mean((o - m) * (o - m), axis=-1, keepdims=True)
    o_ref[...] = (o - m) * lax.rsqrt(v + 1e-5) * g[...] + be[...]


BR = 1024


def _dense(p0, p1, c0, c1, x, W, b, Ws, bs, g, be):
    row_spec = pl.BlockSpec((BR, D), lambda i: (i, 0))
    cnt_spec = pl.BlockSpec((BR, CW), lambda i: (i, 0))
    mat_spec = pl.BlockSpec((D, D), lambda i: (0, 0))
    vec_spec = pl.BlockSpec((1, D), lambda i: (0, 0))
    return pl.pallas_call(
        _dense_block,
        grid=(NP // BR,),
        in_specs=[row_spec, row_spec, cnt_spec, cnt_spec, row_spec,
                  mat_spec, vec_spec, mat_spec, vec_spec, vec_spec, vec_spec],
        out_specs=row_spec,
        out_shape=jax.ShapeDtypeStruct((NP, D), jnp.float32),
    )(p0, p1, c0, c1, x, W, b.reshape(1, D), Ws, bs.reshape(1, D),
      g.reshape(1, D), be.reshape(1, D))


def kernel(edges, node_emb, rel_emb, W1, b1, W2, b2, Ws1, bs1, Ws2, bs2,
           g1, be1, g2, be2):
    src = edges[:, 0].reshape(NW, EPW)
    dst = edges[:, 2].reshape(NW, NCHUNK, C)
    x = jnp.pad(node_emb, ((0, NP - N), (0, 0)))
    zrow = jnp.zeros((SB, D), jnp.float32)
    zcnt = jnp.zeros((SB, CW), jnp.float32)
    ones = jnp.ones((C, CW), jnp.float32)

    sc_agg = _make_sc_agg()
    c0, c1 = _make_sc_cnt()(dst, zcnt, ones)
    a0, a1 = sc_agg(x, src, dst, zrow)
    x1 = _dense(a0, a1, c0, c1, x, W1, b1, Ws1, bs1, g1, be1)
    b0, b1_ = sc_agg(x1, src, dst, zrow)
    x2 = _dense(b0, b1_, c0, c1, x1, W2, b2, Ws2, bs2, g2, be2)
    return x2[:N]


# speedup vs baseline: 8.4976x; 1.0023x over previous
import jax
import jax.numpy as jnp
from jax import lax
from jax.experimental import pallas as pl
from jax.experimental.pallas import tpu as pltpu
from jax.experimental.pallas import tpu_sc as plsc

N = 10000
D = 128
E = 320000
NC = 2
NS = 16
NW = NC * NS
NP = 10240
EPW = E // NW
C = 80
NCHUNK = EPW // C
CW = 128
SB = 80
RPT = NP // NS


def _sc_agg_body(x_hbm, src_hbm, dst_hbm, zrow_hbm, out0, out1,
                 sidx, didx, rows, agg_sh, semA, semB):
    cid = lax.axis_index("c")
    sid = lax.axis_index("s")
    wid = sid * NC + cid

    row0 = sid * RPT
    isem = pltpu.async_copy(src_hbm.at[wid], sidx, semA)
    pltpu.sync_copy(dst_hbm.at[wid], didx)
    isem.wait()
    pltpu.sync_copy(zrow_hbm, rows.at[0])
    for part in range(RPT // SB):
        pltpu.sync_copy(rows.at[0], agg_sh.at[pl.ds(row0 + part * SB, SB)])
    plsc.subcore_barrier()

    pltpu.async_copy(x_hbm.at[sidx.at[pl.ds(0, C)]], rows.at[0], semA)

    @pl.loop(0, NCHUNK // 2)
    def _(k):
        j0 = 2 * k
        pltpu.make_async_copy(x_hbm.at[sidx.at[pl.ds(j0 * C, C)]],
                              rows.at[0], semA).wait()
        pltpu.async_copy(x_hbm.at[sidx.at[pl.ds((j0 + 1) * C, C)]],
                         rows.at[1], semB)
        pltpu.sync_copy(rows.at[0], agg_sh.at[didx.at[j0]], add=True)
        pltpu.make_async_copy(x_hbm.at[sidx.at[pl.ds((j0 + 1) * C, C)]],
                              rows.at[1], semB).wait()
        pltpu.async_copy(x_hbm.at[sidx.at[pl.ds((j0 + 2) * C, C)]],
                         rows.at[0], semA)
        pltpu.sync_copy(rows.at[1], agg_sh.at[didx.at[j0 + 1]], add=True)

    pltpu.make_async_copy(x_hbm.at[sidx.at[pl.ds((NCHUNK - 1) * C, C)]],
                          rows.at[0], semA).wait()
    pltpu.sync_copy(rows.at[0], agg_sh.at[didx.at[NCHUNK - 1]], add=True)

    plsc.subcore_barrier()

    @pl.when(cid == 0)
    def _():
        for part in range(RPT // SB):
            r = row0 + part * SB
            pltpu.sync_copy(agg_sh.at[pl.ds(r, SB)], rows.at[0])
            pltpu.sync_copy(rows.at[0], out0.at[pl.ds(r, SB)])

    @pl.when(cid == 1)
    def _():
        for part in range(RPT // SB):
            r = row0 + part * SB
            pltpu.sync_copy(agg_sh.at[pl.ds(r, SB)], rows.at[0])
            pltpu.sync_copy(rows.at[0], out1.at[pl.ds(r, SB)])


def _make_sc_agg():
    mesh = plsc.VectorSubcoreMesh(core_axis_name="c", subcore_axis_name="s")
    return pl.kernel(
        _sc_agg_body,
        out_type=(jax.ShapeDtypeStruct((NP, D), jnp.float32),
                  jax.ShapeDtypeStruct((NP, D), jnp.float32)),
        mesh=mesh,
        scratch_types=(
            pltpu.VMEM((EPW,), jnp.int32),
            pltpu.VMEM((NCHUNK, C), jnp.int32),
            pltpu.VMEM((2, C, D), jnp.float32),
            pltpu.VMEM_SHARED((NP, D), jnp.float32),
            pltpu.SemaphoreType.DMA,
            pltpu.SemaphoreType.DMA,
        ),
    )


def _sc_cnt_body(dst_hbm, zcnt_hbm, ones_hbm, cnt0, cnt1,
                 didx, ones_v, cstage, cnt_sh, csem):
    cid = lax.axis_index("c")
    sid = lax.axis_index("s")
    wid = sid * NC + cid

    row0 = sid * RPT
    pltpu.sync_copy(dst_hbm.at[wid], didx)
    pltpu.sync_copy(zcnt_hbm, cstage)
    pltpu.sync_copy(ones_hbm, ones_v)
    for part in range(RPT // SB):
        pltpu.sync_copy(cstage, cnt_sh.at[pl.ds(row0 + part * SB, SB)])
    plsc.subcore_barrier()

    @pl.loop(0, NCHUNK)
    def _(j):
        pltpu.async_copy(ones_v, cnt_sh.at[didx.at[j]], csem, add=True)

    @pl.loop(0, NCHUNK)
    def _(j):
        pltpu.make_async_copy(ones_v, cnt_sh.at[didx.at[0]], csem).wait()

    plsc.subcore_barrier()

    @pl.when(cid == 0)
    def _():
        for part in range(RPT // SB):
            r = row0 + part * SB
            pltpu.sync_copy(cnt_sh.at[pl.ds(r, SB)], cstage)
            pltpu.sync_copy(cstage, cnt0.at[pl.ds(r, SB)])

    @pl.when(cid == 1)
    def _():
        for part in range(RPT // SB):
            r = row0 + part * SB
            pltpu.sync_copy(cnt_sh.at[pl.ds(r, SB)], cstage)
            pltpu.sync_copy(cstage, cnt1.at[pl.ds(r, SB)])


def _make_sc_cnt():
    mesh = plsc.VectorSubcoreMesh(core_axis_name="c", subcore_axis_name="s")
    return pl.kernel(
        _sc_cnt_body,
        out_type=(jax.ShapeDtypeStruct((NP, CW), jnp.float32),
                  jax.ShapeDtypeStruct((NP, CW), jnp.float32)),
        mesh=mesh,
        scratch_types=(
            pltpu.VMEM((NCHUNK, C), jnp.int32),
            pltpu.VMEM((C, CW), jnp.float32),
            pltpu.VMEM((SB, CW), jnp.float32),
            pltpu.VMEM_SHARED((NP, CW), jnp.float32),
            pltpu.SemaphoreType.DMA,
        ),
    )


def _dense_block(p0, p1, c0, c1, x, W, b, Ws, bs, g, be, o_ref):
    agg = p0[...] + p1[...]
    cnt = c0[:, 0:1] + c1[:, 0:1]
    mean = agg / jnp.maximum(cnt, 1.0)
    dn = (((1,), (1,)), ((), ()))
    h = lax.dot_general(mean, W[...], dn, preferred_element_type=jnp.float32)
    h = h + b[...] * jnp.where(cnt > 0.0, 1.0, 0.0)
    h = jnp.maximum(h, 0.0)
    o = h + lax.dot_general(x[...], Ws[...], dn,
                            preferred_element_type=jnp.float32) + bs[...]
    m = jnp.mean(o, axis=-1, keepdims=True)
    v = jnp.mean((o - m) * (o - m), axis=-1, keepdims=True)
    o_ref[...] = (o - m) * lax.rsqrt(v + 1e-5) * g[...] + be[...]


BR = 1024


def _dense(p0, p1, c0, c1, x, W, b, Ws, bs, g, be):
    row_spec = pl.BlockSpec((BR, D), lambda i: (i, 0))
    cnt_spec = pl.BlockSpec((BR, CW), lambda i: (i, 0))
    mat_spec = pl.BlockSpec((D, D), lambda i: (0, 0))
    vec_spec = pl.BlockSpec((1, D), lambda i: (0, 0))
    return pl.pallas_call(
        _dense_block,
        grid=(NP // BR,),
        in_specs=[row_spec, row_spec, cnt_spec, cnt_spec, row_spec,
                  mat_spec, vec_spec, mat_spec, vec_spec, vec_spec, vec_spec],
        out_specs=row_spec,
        out_shape=jax.ShapeDtypeStruct((NP, D), jnp.float32),
    )(p0, p1, c0, c1, x, W, b.reshape(1, D), Ws, bs.reshape(1, D),
      g.reshape(1, D), be.reshape(1, D))


def kernel(edges, node_emb, rel_emb, W1, b1, W2, b2, Ws1, bs1, Ws2, bs2,
           g1, be1, g2, be2):
    src = edges[:, 0].reshape(NW, EPW)
    dst = edges[:, 2].reshape(NW, NCHUNK, C)
    x = jnp.pad(node_emb, ((0, NP - N), (0, 0)))
    zrow = jnp.zeros((SB, D), jnp.float32)
    zcnt = jnp.zeros((SB, CW), jnp.float32)
    ones = jnp.ones((C, CW), jnp.float32)

    sc_agg = _make_sc_agg()
    c0, c1 = _make_sc_cnt()(dst, zcnt, ones)
    a0, a1 = sc_agg(x, src, dst, zrow)
    x1 = _dense(a0, a1, c0, c1, x, W1, b1, Ws1, bs1, g1, be1)
    b0, b1_ = sc_agg(x1, src, dst, zrow)
    x2 = _dense(b0, b1_, c0, c1, x1, W2, b2, Ws2, bs2, g2, be2)
    return x2[:N]
